# baseline (speedup 1.0000x reference)
import functools

import jax
import jax.numpy as jnp
from jax import lax
from jax.experimental import pallas as pl
from jax.experimental.pallas import tpu as pltpu
from jax.experimental.pallas import tpu_sc as plsc

_B = 256
_L = 16
_NC = 2
_NS = 16
_NW = _NC * _NS

_N = 100000
_E = 3200000
_DF = 128
_DE = 16

_ST = 2
_LT = _E // 128
_T_LT = 4
_N_CH = _LT // _T_LT
_K_E = (_N_CH + _NW - 1) // _NW
_CH_EDGES = _T_LT * 128

_TB_N = 16
_NT_N = _N // _TB_N
_K_N = (_NT_N + _NW - 1) // _NW


def _sc_body(x_hbm, vidx_hbm, eb_hbm, eidx_hbm,
             npart_hbm, epart_hbm, cntn_hbm, cnte_hbm,
             acc_n, acc_e, cnt_n, cnt_e,
             ebuf0, ebuf1, eibuf0, eibuf1, esem0, esem1,
             nbuf0, nbuf1, nibuf0, nibuf1, nsem0, nsem1):
    wid = lax.axis_index("c") * _NS + lax.axis_index("s")
    ones = jnp.ones((_L,), jnp.float32)
    zrow = jnp.zeros((_L,), jnp.float32)
    iota = lax.iota(jnp.int32, _L)
    ebufs, eibufs, esems = (ebuf0, ebuf1), (eibuf0, eibuf1), (esem0, esem1)
    nbufs, nibufs, nsems = (nbuf0, nbuf1), (nibuf0, nibuf1), (nsem0, nsem1)

    def zero_row(r, _):
        cnt_n[r] = zrow
        for c in range(_DE):
            acc_e[c, pl.ds(r * _L, _L)] = zrow
        cnt_e[pl.ds(r * _L, _L)] = zrow
        for c in range(_DF // _L):
            acc_n[r, pl.ds(c * _L, _L)] = zrow
        return 0
    lax.fori_loop(0, _B, zero_row, 0)

    def fire_e(k, b):
        ch = wid + _NW * k

        @pl.when(ch < _N_CH)
        def _():
            lt0 = ch * _T_LT
            for st in range(_ST):
                pltpu.async_copy(eb_hbm.at[st, pl.ds(lt0, _T_LT)],
                                 ebufs[b].at[st], esems[b])
            pltpu.async_copy(eidx_hbm.at[pl.ds(lt0 * 128, _CH_EDGES)],
                             eibufs[b], esems[b])

    def proc_e(k, b):
        ch = wid + _NW * k

        @pl.when(ch < _N_CH)
        def _():
            for st in range(_ST):
                pltpu.make_async_copy(eb_hbm.at[st, pl.ds(0, _T_LT)],
                                      ebufs[b].at[st], esems[b]).wait()
            pltpu.make_async_copy(eidx_hbm.at[pl.ds(0, _CH_EDGES)],
                                  eibufs[b], esems[b]).wait()
            ebuf, eibuf = ebufs[b], eibufs[b]

            def grp(gp, _):
                sidxs, valss = [], []
                for h in range(2):
                    g = gp * 2 + h
                    t = lax.shift_right_logical(g, 3)
                    m16 = lax.mul(lax.bitwise_and(g, 7), _L)
                    segv = eibuf[pl.ds(t * 128 + m16, _L)]
                    sidxs.append(segv * _L + iota)
                    valss.append([ebuf[st, t, sl, pl.ds(m16, _L)]
                                  for st in range(_ST) for sl in range(8)])
                for h in range(2):
                    plsc.addupdate_scatter(cnt_e, [sidxs[h]], ones)
                    for c in range(_DE):
                        plsc.addupdate_scatter(acc_e.at[c], [sidxs[h]],
                                               valss[h][c])
                return 0
            lax.fori_loop(0, _T_LT * 4, grp, 0)

    fire_e(0, 0)

    def e_pair(kp, _):
        k0 = 2 * kp
        fire_e(k0 + 1, 1)
        proc_e(k0, 0)
        fire_e(k0 + 2, 0)
        proc_e(k0 + 1, 1)
        return 0
    lax.fori_loop(0, _K_E // 2, e_pair, 0)

    def fire_n(k, b):
        t = wid + _NW * k

        @pl.when(t < _NT_N)
        def _():
            base = t * _TB_N
            pltpu.async_copy(x_hbm.at[pl.ds(base, _TB_N)], nbufs[b], nsems[b])
            pltpu.async_copy(vidx_hbm.at[pl.ds(base, _TB_N)], nibufs[b],
                             nsems[b])

    def proc_n(k, b):
        t = wid + _NW * k

        @pl.when(t < _NT_N)
        def _():
            pltpu.make_async_copy(x_hbm.at[pl.ds(0, _TB_N)], nbufs[b],
                                  nsems[b]).wait()
            pltpu.make_async_copy(vidx_hbm.at[pl.ds(0, _TB_N)], nibufs[b],
                                  nsems[b]).wait()
            nbuf, nibuf = nbufs[b], nibufs[b]
            segv = nibuf[pl.ds(0, _L)]
            s0 = segv[0]
            s1 = segv[_L - 1]

            def fast():
                for c in range(_DF // _L):
                    sl = pl.ds(c * _L, _L)
                    acc = nbuf[0, sl]
                    for j in range(1, _L):
                        acc = acc + nbuf[j, sl]
                    plsc.addupdate(acc_n.at[s0, sl], acc)
                plsc.addupdate(cnt_n.at[s0], jnp.full((_L,), float(_L),
                                                      jnp.float32))

            def slow():
                for j in range(_L):
                    seg = segv[j]
                    for c in range(_DF // _L):
                        sl = pl.ds(c * _L, _L)
                        plsc.addupdate(acc_n.at[seg, sl], nbuf[j, sl])
                    plsc.addupdate(cnt_n.at[seg], ones)

            lax.cond(s0 == s1, fast, slow)

    fire_n(0, 0)

    def n_pair(kp, _):
        k0 = 2 * kp
        fire_n(k0 + 1, 1)
        proc_n(k0, 0)
        fire_n(k0 + 2, 0)
        proc_n(k0 + 1, 1)
        return 0
    lax.fori_loop(0, _K_N // 2, n_pair, 0)

    pltpu.sync_copy(acc_n, npart_hbm.at[wid])
    pltpu.sync_copy(acc_e, epart_hbm.at[wid])
    pltpu.sync_copy(cnt_n, cntn_hbm.at[wid])
    pltpu.sync_copy(cnt_e, cnte_hbm.at[wid])


@jax.jit
def _sc_segment_sums(x, v_indices, edge_bytes, e_indices):
    mesh = plsc.VectorSubcoreMesh(core_axis_name="c", subcore_axis_name="s")
    f32 = jnp.float32
    return pl.kernel(
        _sc_body,
        out_type=(
            jax.ShapeDtypeStruct((_NW, _B, _DF), f32),
            jax.ShapeDtypeStruct((_NW, _DE, _L * _B), f32),
            jax.ShapeDtypeStruct((_NW, _B, _L), f32),
            jax.ShapeDtypeStruct((_NW, _L * _B), f32),
        ),
        mesh=mesh,
        compiler_params=pltpu.CompilerParams(use_tc_tiling_on_sc=False,
                                             needs_layout_passes=False),
        scratch_types=[
            pltpu.VMEM((_B, _DF), f32),
            pltpu.VMEM((_DE, _L * _B), f32),
            pltpu.VMEM((_B, _L), f32),
            pltpu.VMEM((_L * _B,), f32),
            pltpu.VMEM((_ST, _T_LT, 8, 128), f32),
            pltpu.VMEM((_ST, _T_LT, 8, 128), f32),
            pltpu.VMEM((_CH_EDGES,), jnp.int32),
            pltpu.VMEM((_CH_EDGES,), jnp.int32),
            pltpu.SemaphoreType.DMA,
            pltpu.SemaphoreType.DMA,
            pltpu.VMEM((_TB_N, _DF), f32),
            pltpu.VMEM((_TB_N, _DF), f32),
            pltpu.VMEM((_TB_N,), jnp.int32),
            pltpu.VMEM((_TB_N,), jnp.int32),
            pltpu.SemaphoreType.DMA,
            pltpu.SemaphoreType.DMA,
        ],
    )(x, v_indices, edge_bytes, e_indices)


def _finish_body(npart, epart, cn, ce, u, w1, b1, w2, b2, out):
    ns = jnp.sum(npart[...], axis=0)
    fold = (lax.broadcasted_iota(jnp.int32, (_L * _B, _B), 0) // _L
            == lax.broadcasted_iota(jnp.int32, (_L * _B, _B), 1)
            ).astype(jnp.float32)
    esT = jnp.dot(jnp.sum(epart[...], axis=0), fold,
                  preferred_element_type=jnp.float32)
    cnv = jnp.sum(cn[...], axis=0)[:, 0:1]
    cev = jnp.dot(jnp.sum(ce[...], axis=0).reshape(1, _L * _B), fold,
                  preferred_element_type=jnp.float32)
    nm = ns / jnp.maximum(cnv, 1.0)
    emT = esT / jnp.maximum(cev, 1.0)
    f32 = jnp.float32
    h = (jnp.dot(u[...], w1[0:64, :], preferred_element_type=f32)
         + jnp.dot(nm, w1[64:192, :], preferred_element_type=f32)
         + lax.dot_general(emT, w1[192:208, :], (((0,), (0,)), ((), ())),
                           preferred_element_type=f32)
         + b1[...])
    h = jnp.maximum(h, 0.0)
    out[...] = jnp.dot(h, w2[...], preferred_element_type=f32) + b2[...]


@jax.jit
def _tc_finish(npart, epart, cn, ce, u, w1, b1, w2, b2):
    return pl.pallas_call(
        _finish_body,
        out_shape=jax.ShapeDtypeStruct((_B, 64), jnp.float32),
    )(npart, epart, cn, ce, u, w1, b1, w2, b2)


def kernel(x, edge_attr, u, v_indices, e_indices, W1, b1, W2, b2):
    eb = edge_attr.T.reshape(_ST, 8, _LT, 128).transpose(0, 2, 1, 3)
    npart, epart, cn, ce = _sc_segment_sums(
        x, v_indices.astype(jnp.int32), eb, e_indices.astype(jnp.int32))
    return _tc_finish(npart, epart, cn, ce, u, W1,
                      b1.reshape(1, -1), W2, b2.reshape(1, -1))
